# baseline (speedup 1.0000x reference)
import functools
import math

import jax
import jax.numpy as jnp
from jax import lax
from jax.experimental import pallas as pl
from jax.experimental.pallas import tpu as pltpu
from jax.experimental.pallas import tpu_sc as plsc

D_MODEL = 64
VOCAB = 1000000
ROWS = 4096
COLS = 200
B = ROWS * COLS
SCALE = math.sqrt(D_MODEL)
LANES = 16

NW = 32
NBLK = COLS * (ROWS // 128)
BPW = NBLK // NW

_mesh = plsc.VectorSubcoreMesh(core_axis_name="c", subcore_axis_name="s")


@functools.partial(
    pl.kernel,
    mesh=_mesh,
    compiler_params=pltpu.CompilerParams(use_tc_tiling_on_sc=False,
                                         needs_layout_passes=False),
    out_type=jax.ShapeDtypeStruct((COLS, 8, ROWS // 128, 8, 128),
                                  jnp.float32),
    scratch_types=[
        pltpu.VMEM((BPW, 128), jnp.int32),
        pltpu.VMEM((128, D_MODEL), jnp.float32),
        pltpu.VMEM((128, D_MODEL), jnp.float32),
        pltpu.VMEM((8, 8, 128), jnp.float32),
        pltpu.VMEM((8, 8, 128), jnp.float32),
        pltpu.SemaphoreType.DMA,
        pltpu.SemaphoreType.DMA,
        pltpu.SemaphoreType.DMA,
        pltpu.SemaphoreType.DMA,
    ],
)
def _emb_gather(tbl_hbm, idx_hbm, out_hbm, idx_v, gb0, gb1, wb0, wb1,
                gs0, gs1, ws0, ws1):
    wid = lax.axis_index("s") * 2 + lax.axis_index("c")
    tbase = wid * BPW
    gbufs, wbufs = (gb0, gb1), (wb0, wb1)
    gsems, wsems = (gs0, gs1), (ws0, ws1)

    pltpu.sync_copy(idx_hbm.at[pl.ds(tbase, BPW)], idx_v)

    lanes = lax.iota(jnp.int32, LANES)

    def start_gather(i, gb, gs):
        pltpu.async_copy(tbl_hbm.at[idx_v.at[i]], gb, gs)

    def wait_gather(i, gb, gs):
        pltpu.make_async_copy(tbl_hbm.at[idx_v.at[i]], gb, gs).wait()

    def out_dst(i, wb, ws):
        t = tbase + i
        s = t // (ROWS // 128)
        bb = t % (ROWS // 128)
        return pltpu.make_async_copy(wb, out_hbm.at[s, :, bb], ws)

    for b in range(2):
        start_gather(b, gbufs[b], gsems[b])

    def body(jj, carry):
        for b in range(2):
            i = 2 * jj + b
            gb, wb = gbufs[b], wbufs[b]
            wait_gather(i, gb, gsems[b])

            @pl.when(i >= 2)
            def _():
                out_dst(i - 2, wb, wsems[b]).wait()

            @plsc.parallel_loop(0, D_MODEL, 1, unroll=2)
            def _(f):
                fcol = jnp.full((LANES,), 0, jnp.int32) + f
                for g in range(128 // LANES):
                    rows = lanes + (g * LANES)
                    v = plsc.load_gather(gb, [rows, fcol])
                    wb[f // 8, f % 8, pl.ds(g * LANES, LANES)] = v * SCALE

            @pl.when(i + 2 < BPW)
            def _():
                start_gather(i + 2, gb, gsems[b])

            out_dst(i, wb, wsems[b]).start()
        return carry

    lax.fori_loop(0, BPW // 2, body, 0)

    for b in range(2):
        out_dst(BPW - 2 + b, wbufs[b], wsems[b]).wait()


def kernel(x, lut):
    idx = jnp.swapaxes(x, 0, 1).reshape(NBLK, 128).astype(jnp.int32)
    out5 = _emb_gather(lut, idx)
    return out5.transpose(2, 4, 0, 1, 3).reshape(ROWS, COLS, D_MODEL)
